# baseline (speedup 1.0000x reference)
import functools
import jax
import jax.numpy as jnp
from jax import lax
from jax.experimental import pallas as pl
from jax.experimental.pallas import tpu as pltpu
from jax.experimental.pallas import tpu_sc as plsc

N = 20000
NPAD = 20480
NW = 16
CH = NPAD // NW
SLOT = 256
K = 2000
KPAD = 2560
T0 = 0.887
IOU_THRESH = 0.7
MAX_OUT = 300


def _sc_select_body(s_hbm, x1_hbm, y1_hbm, x2_hbm, y2_hbm,
                    o_s, o_x1, o_y1, o_x2, o_y2, o_cnt,
                    ch, co, vec16, sh, sh_cnt, st, ds, cl):
    cid = lax.axis_index("c")
    wid = lax.axis_index("s")

    @pl.when(cid == 0)
    def _():
        base = wid * CH
        in_refs = (s_hbm, x1_hbm, y1_hbm, x2_hbm, y2_hbm)
        for p in range(5):
            pltpu.sync_copy(in_refs[p].at[pl.ds(base, CH)], ch[p])

        lane = lax.iota(jnp.int32, 16)

        def body(i, off):
            sv = ch[0][pl.ds(i * 16, 16)]
            m = sv > T0
            mi = m.astype(jnp.int32)
            rank = plsc.cumsum(mi) - mi
            idx = jnp.where(m, off + rank, CH + lane)
            plsc.store_scatter(co[0], [idx], sv)
            for p in range(1, 5):
                plsc.store_scatter(co[p], [idx], ch[p][pl.ds(i * 16, 16)])
            return off + jnp.max(plsc.all_reduce_population_count(m))

        cnt_w = lax.fori_loop(0, CH // 16, body, jnp.int32(0))

        for p in range(5):
            pltpu.sync_copy(co[p].at[pl.ds(0, SLOT)],
                            sh[p].at[pl.ds(wid * SLOT, SLOT)])
        vec16[...] = jnp.broadcast_to(cnt_w, (16,))
        pltpu.sync_copy(vec16, sh_cnt.at[pl.ds(wid * 16, 16)])
        plsc.subcore_barrier()

        @pl.when(wid == 0)
        def _merge():
            pltpu.sync_copy(sh_cnt, cl)
            for p in range(5):
                pltpu.sync_copy(sh[p], st[p])

            def initb(g, _):
                ds[0][pl.ds(g * 16, 16)] = jnp.full((16,), -1e9, jnp.float32)
                for p in range(1, 5):
                    ds[p][pl.ds(g * 16, 16)] = jnp.zeros((16,), jnp.float32)
                return 0

            lax.fori_loop(0, (NW * SLOT) // 16, initb, 0)

            def mbody(v, off):
                cv = jnp.minimum(jnp.max(cl[pl.ds(v * 16, 16)]),
                                 jnp.int32(SLOT))

                def inner(g, off2):
                    lane2 = lax.iota(jnp.int32, 16)
                    m2 = (lane2 + g * 16) < cv
                    mi2 = m2.astype(jnp.int32)
                    rank2 = plsc.cumsum(mi2) - mi2
                    idx2 = jnp.where(m2, off2 + rank2, NW * SLOT + lane2)
                    for p in range(5):
                        plsc.store_scatter(
                            ds[p], [idx2],
                            st[p][pl.ds(v * SLOT + g * 16, 16)])
                    return off2 + jnp.max(
                        plsc.all_reduce_population_count(m2))

                return lax.fori_loop(0, SLOT // 16, inner, off)

            lax.fori_loop(0, NW, mbody, jnp.int32(0))

            def cbody(v, c):
                tot, ovf = c
                cv = jnp.max(cl[pl.ds(v * 16, 16)])
                return tot + cv, ovf | (cv > SLOT)

            tot, ovf = lax.fori_loop(0, NW, cbody,
                                     (jnp.int32(0), jnp.bool_(False)))
            n_out = jnp.where(ovf, jnp.int32(99999), tot)

            out_refs = (o_s, o_x1, o_y1, o_x2, o_y2)
            for p in range(5):
                pltpu.sync_copy(ds[p].at[pl.ds(0, KPAD)], out_refs[p])
            vec16[...] = jnp.broadcast_to(n_out, (16,))
            pltpu.sync_copy(vec16, o_cnt)


def _sc_select(scores_p, x1, y1, x2, y2):
    mesh = plsc.VectorSubcoreMesh(core_axis_name="c", subcore_axis_name="s")
    f32 = jnp.float32
    plane = jax.ShapeDtypeStruct((KPAD,), f32)
    kern = pl.kernel(
        lambda *refs: _sc_select_body(
            *refs[:11],
            ch=refs[11:16], co=refs[16:21], vec16=refs[21],
            sh=refs[22:27], sh_cnt=refs[27], st=refs[28:33],
            ds=refs[33:38], cl=refs[38]),
        out_type=[plane, plane, plane, plane, plane,
                  jax.ShapeDtypeStruct((16,), jnp.int32)],
        mesh=mesh,
        compiler_params=pltpu.CompilerParams(needs_layout_passes=False),
        scratch_types=(
            [pltpu.VMEM((CH,), f32)] * 5
            + [pltpu.VMEM((CH + 16,), f32)] * 5
            + [pltpu.VMEM((16,), jnp.int32)]
            + [pltpu.VMEM_SHARED((NW * SLOT,), f32)] * 5
            + [pltpu.VMEM_SHARED((NW * 16,), jnp.int32)]
            + [pltpu.VMEM((NW * SLOT,), f32)] * 5
            + [pltpu.VMEM((NW * SLOT + 16,), f32)] * 5
            + [pltpu.VMEM((NW * 16,), jnp.int32)]
        ),
    )
    return kern(scores_p, x1, y1, x2, y2)


RCH = 320


def _nms_kernel(cols_ref, rows_ref, keep_ref, m_ref):
    x1r = rows_ref[0:1, :]
    y1r = rows_ref[1:2, :]
    x2r = rows_ref[2:3, :]
    y2r = rows_ref[3:4, :]
    sr = rows_ref[4:5, :]
    area_r = (x2r - x1r) * (y2r - y1r)

    for c in range(KPAD // RCH):
        tile = cols_ref[pl.ds(c * RCH, RCH), :]
        x1c = tile[:, 0:1]
        y1c = tile[:, 1:2]
        x2c = tile[:, 2:3]
        y2c = tile[:, 3:4]
        sc = tile[:, 4:5]
        area_c = (x2c - x1c) * (y2c - y1c)
        xx1 = jnp.maximum(x1c, x1r)
        yy1 = jnp.maximum(y1c, y1r)
        xx2 = jnp.minimum(x2c, x2r)
        yy2 = jnp.minimum(y2c, y2r)
        iw = jnp.clip(xx2 - xx1, 0.0)
        ih = jnp.clip(yy2 - yy1, 0.0)
        inter = iw * ih
        union = area_c + area_r - inter
        iou = inter / jnp.maximum(union, 1e-9)

        gi = lax.broadcasted_iota(jnp.int32, (RCH, KPAD), 0) + c * RCH
        gj = lax.broadcasted_iota(jnp.int32, (RCH, KPAD), 1)
        pri = (sc > sr) | ((sc == sr) & (gi < gj))
        m_ref[pl.ds(c * RCH, RCH), :] = (
            (iou > IOU_THRESH) & pri).astype(jnp.bfloat16)

    def cond(c):
        return c[1]

    def body(c):
        k, _ = c
        cnt = lax.dot_general(
            k.astype(jnp.bfloat16), m_ref[:, :],
            (((1,), (0,)), ((), ())),
            preferred_element_type=jnp.float32,
        )
        k_new = (cnt == 0.0).astype(jnp.float32)
        return k_new, jnp.any(k_new != k)

    k0 = jnp.ones((1, KPAD), jnp.float32)
    k_final, _ = lax.while_loop(cond, body, (k0, True))
    keep_ref[:, :] = k_final


def _nms_keep(cols, rows):
    return pl.pallas_call(
        _nms_kernel,
        out_shape=jax.ShapeDtypeStruct((1, KPAD), jnp.float32),
        scratch_shapes=[pltpu.VMEM((KPAD, KPAD), jnp.bfloat16)],
    )(cols, rows)


def _nms_select(s, x1, y1, x2, y2):
    cols = jnp.stack([x1, y1, x2, y2, s], axis=1)
    rows = cols.T
    keep = _nms_keep(cols, rows)[0]
    keep_b = keep > 0.5
    masked = jnp.where(keep_b, s, -1e9)
    final_scores, final_idx = lax.top_k(masked, MAX_OUT)
    final_boxes = jnp.take(cols[:, :4], final_idx, axis=0)
    return final_boxes, final_scores, keep_b


def kernel(boxes, scores):
    scores_p = jnp.concatenate(
        [scores, jnp.full((NPAD - N,), -1.0, jnp.float32)])
    bpad = jnp.concatenate([boxes, jnp.zeros((NPAD - N, 4), jnp.float32)])
    planes = [bpad[:, i] for i in range(4)]
    s_sel, x1s, y1s, x2s, y2s, n_sel_v = _sc_select(scores_p, *planes)
    n_sel = n_sel_v[0]

    fb, fs, keep_b = _nms_select(s_sel, x1s, y1s, x2s, y2s)

    n_kept = jnp.sum(keep_b.astype(jnp.int32))
    n_kept_valid = n_kept - (KPAD - jnp.minimum(n_sel, KPAD))
    extras = n_sel - K
    ok = ((n_sel >= K) & (n_sel <= KPAD)
          & (n_kept_valid - extras >= MAX_OUT))

    def fast(_):
        return fb, fs

    def slow(_):
        top_scores, order = lax.top_k(scores, K)
        top_boxes = jnp.take(boxes, order, axis=0)
        sp = jnp.concatenate(
            [top_scores, jnp.full((KPAD - K,), -1e9, jnp.float32)])
        bp = jnp.concatenate(
            [top_boxes, jnp.zeros((KPAD - K, 4), jnp.float32)])
        b2, s2, _ = _nms_select(sp, bp[:, 0], bp[:, 1], bp[:, 2], bp[:, 3])
        return b2, s2

    return lax.cond(ok, fast, slow, None)


# speedup vs baseline: 1.0086x; 1.0086x over previous
import jax
import jax.numpy as jnp
from jax import lax
from jax.experimental import pallas as pl
from jax.experimental.pallas import tpu as pltpu
from jax.experimental.pallas import tpu_sc as plsc

N = 20000
NPAD = 20480
NW = 16
CH = NPAD // NW
SLOT = 256
K = 2000
KPAD = 2560
T0 = 0.887
IOU_THRESH = 0.7
MAX_OUT = 300
MERGED = NW * SLOT


def _sc_select_body(s_hbm, x1_hbm, y1_hbm, x2_hbm, y2_hbm,
                    o_cols, o_rows, o_cnt,
                    ch, co, vec16, sh, sh_cnt, st, dsc, dsr, cl):
    cid = lax.axis_index("c")
    wid = lax.axis_index("s")

    @pl.when(cid == 0)
    def _():
        base = wid * CH
        in_refs = (s_hbm, x1_hbm, y1_hbm, x2_hbm, y2_hbm)
        for p in range(5):
            pltpu.sync_copy(in_refs[p].at[pl.ds(base, CH)], ch[p])

        lane = lax.iota(jnp.int32, 16)

        def body(i, off):
            sv = ch[0][pl.ds(i * 16, 16)]
            m = sv > T0
            mi = m.astype(jnp.int32)
            rank = plsc.cumsum(mi) - mi
            idx = jnp.where(m, off + rank, CH + lane)
            plsc.store_scatter(co[0], [idx], sv)
            for p in range(1, 5):
                plsc.store_scatter(co[p], [idx], ch[p][pl.ds(i * 16, 16)])
            return off + jnp.max(plsc.all_reduce_population_count(m))

        cnt_w = lax.fori_loop(0, CH // 16, body, jnp.int32(0))

        for p in range(5):
            pltpu.sync_copy(co[p].at[pl.ds(0, SLOT)],
                            sh[p].at[pl.ds(wid * SLOT, SLOT)])
        vec16[...] = jnp.broadcast_to(cnt_w, (16,))
        pltpu.sync_copy(vec16, sh_cnt.at[pl.ds(wid * 16, 16)])
        plsc.subcore_barrier()

        @pl.when(wid == 0)
        def _merge():
            pltpu.sync_copy(sh_cnt, cl)
            for p in range(5):
                pltpu.sync_copy(sh[p], st[p])

            def initb(g, _):
                iv = lax.iota(jnp.int32, 16) + g * 16
                dsc[pl.ds(g * 16, 16)] = jnp.where(
                    iv % 5 == 4, -1e9, 0.0).astype(jnp.float32)
                return 0

            lax.fori_loop(0, (KPAD * 5) // 16, initb, 0)

            def initr(g, _):
                dsr[0][pl.ds(g * 16, 16)] = jnp.full((16,), -1e9,
                                                     jnp.float32)
                for p in range(1, 5):
                    dsr[p][pl.ds(g * 16, 16)] = jnp.zeros((16,),
                                                          jnp.float32)
                return 0

            lax.fori_loop(0, KPAD // 16, initr, 0)

            def mbody(v, off):
                cv = jnp.minimum(jnp.max(cl[pl.ds(v * 16, 16)]),
                                 jnp.int32(SLOT))

                def inner(g, off2):
                    lane2 = lax.iota(jnp.int32, 16)
                    m2 = (lane2 + g * 16) < cv
                    mi2 = m2.astype(jnp.int32)
                    rank2 = plsc.cumsum(mi2) - mi2
                    pos = off2 + rank2
                    m3 = m2 & (pos < KPAD)
                    posr = jnp.where(m3, pos, MERGED + lane2)
                    for p in range(5):
                        cp = (p + 4) % 5
                        x = st[p][pl.ds(v * SLOT + g * 16, 16)]
                        posc = jnp.where(m3, pos * 5 + cp,
                                         KPAD * 5 + lane2 * 5 + cp)
                        plsc.store_scatter(dsc, [posc], x)
                        plsc.store_scatter(dsr[p], [posr], x)
                    return off2 + jnp.max(
                        plsc.all_reduce_population_count(m2))

                return lax.fori_loop(0, SLOT // 16, inner, off)

            lax.fori_loop(0, NW, mbody, jnp.int32(0))

            def cbody(v, c):
                tot, ovf = c
                cv = jnp.max(cl[pl.ds(v * 16, 16)])
                return tot + cv, ovf | (cv > SLOT)

            tot, ovf = lax.fori_loop(0, NW, cbody,
                                     (jnp.int32(0), jnp.bool_(False)))
            n_out = jnp.where(ovf, jnp.int32(99999), tot)

            pltpu.sync_copy(dsc.at[pl.ds(0, KPAD * 5)], o_cols)
            for p in range(5):
                cp = (p + 4) % 5
                pltpu.sync_copy(dsr[p].at[pl.ds(0, KPAD)],
                                o_rows.at[pl.ds(cp * KPAD, KPAD)])
            vec16[...] = jnp.broadcast_to(n_out, (16,))
            pltpu.sync_copy(vec16, o_cnt)


def _sc_select(scores_p, x1, y1, x2, y2):
    mesh = plsc.VectorSubcoreMesh(core_axis_name="c", subcore_axis_name="s")
    f32 = jnp.float32
    kern = pl.kernel(
        lambda *refs: _sc_select_body(
            *refs[:8],
            ch=refs[8:13], co=refs[13:18], vec16=refs[18],
            sh=refs[19:24], sh_cnt=refs[24], st=refs[25:30],
            dsc=refs[30], dsr=refs[31:36], cl=refs[36]),
        out_type=[jax.ShapeDtypeStruct((KPAD * 5,), f32),
                  jax.ShapeDtypeStruct((5 * KPAD,), f32),
                  jax.ShapeDtypeStruct((16,), jnp.int32)],
        mesh=mesh,
        compiler_params=pltpu.CompilerParams(needs_layout_passes=False),
        scratch_types=(
            [pltpu.VMEM((CH,), f32)] * 5
            + [pltpu.VMEM((CH + 16,), f32)] * 5
            + [pltpu.VMEM((16,), jnp.int32)]
            + [pltpu.VMEM_SHARED((NW * SLOT,), f32)] * 5
            + [pltpu.VMEM_SHARED((NW * 16,), jnp.int32)]
            + [pltpu.VMEM((NW * SLOT,), f32)] * 5
            + [pltpu.VMEM((KPAD * 5 + 80,), f32)]
            + [pltpu.VMEM((MERGED + 16,), f32)] * 5
            + [pltpu.VMEM((NW * 16,), jnp.int32)]
        ),
    )
    return kern(scores_p, x1, y1, x2, y2)


RCH = 320


def _nms_kernel(cols_ref, rows_ref, nsel_ref, masked_ref, stats_ref, m_ref):
    x1r = rows_ref[0:1, :]
    y1r = rows_ref[1:2, :]
    x2r = rows_ref[2:3, :]
    y2r = rows_ref[3:4, :]
    sr = rows_ref[4:5, :]
    area_r = (x2r - x1r) * (y2r - y1r)

    for c in range(KPAD // RCH):
        tile = cols_ref[pl.ds(c * RCH, RCH), :]
        x1c = tile[:, 0:1]
        y1c = tile[:, 1:2]
        x2c = tile[:, 2:3]
        y2c = tile[:, 3:4]
        sc = tile[:, 4:5]
        area_c = (x2c - x1c) * (y2c - y1c)
        xx1 = jnp.maximum(x1c, x1r)
        yy1 = jnp.maximum(y1c, y1r)
        xx2 = jnp.minimum(x2c, x2r)
        yy2 = jnp.minimum(y2c, y2r)
        iw = jnp.clip(xx2 - xx1, 0.0)
        ih = jnp.clip(yy2 - yy1, 0.0)
        inter = iw * ih
        union = area_c + area_r - inter
        iou = inter / jnp.maximum(union, 1e-9)

        gi = lax.broadcasted_iota(jnp.int32, (RCH, KPAD), 0) + c * RCH
        gj = lax.broadcasted_iota(jnp.int32, (RCH, KPAD), 1)
        pri = (sc > sr) | ((sc == sr) & (gi < gj))
        m_ref[pl.ds(c * RCH, RCH), :] = (
            (iou > IOU_THRESH) & pri).astype(jnp.bfloat16)

    def cond(c):
        return c[1]

    def body(c):
        k, _ = c
        cnt = lax.dot_general(
            k.astype(jnp.bfloat16), m_ref[:, :],
            (((1,), (0,)), ((), ())),
            preferred_element_type=jnp.float32,
        )
        k_new = (cnt == 0.0).astype(jnp.float32)
        return k_new, jnp.any(k_new != k)

    k0 = jnp.ones((1, KPAD), jnp.float32)
    k_final, _ = lax.while_loop(cond, body, (k0, True))

    masked_ref[:, :] = jnp.where(k_final > 0.5, sr, -1e9)
    nsel = nsel_ref[0, 0]
    gcol = lax.broadcasted_iota(jnp.int32, (1, KPAD), 1)
    kept_valid = jnp.sum(jnp.where((k_final > 0.5) & (gcol < nsel),
                                   1.0, 0.0))
    stats_ref[:, :] = jnp.full((1, 128), kept_valid, jnp.float32)


def _nms_masked(cols, rows, nsel):
    return pl.pallas_call(
        _nms_kernel,
        out_shape=[jax.ShapeDtypeStruct((1, KPAD), jnp.float32),
                   jax.ShapeDtypeStruct((1, 128), jnp.float32)],
        scratch_shapes=[pltpu.VMEM((KPAD, KPAD), jnp.bfloat16)],
    )(cols, rows, nsel)


def kernel(boxes, scores):
    scores_p = jnp.concatenate(
        [scores, jnp.full((NPAD - N,), -1.0, jnp.float32)])
    bpad = jnp.concatenate([boxes, jnp.zeros((NPAD - N, 4), jnp.float32)])
    cols_f, rows_f, n_sel_v = _sc_select(
        scores_p, bpad[:, 0], bpad[:, 1], bpad[:, 2], bpad[:, 3])
    n_sel = n_sel_v[0]
    cols = cols_f.reshape(KPAD, 5)
    rows = rows_f.reshape(5, KPAD)

    masked_f, stats = _nms_masked(cols, rows,
                                  n_sel_v[:1].reshape(1, 1))
    masked = masked_f[0]
    kept_valid = stats[0, 0].astype(jnp.int32)

    extras = n_sel - K
    ok = ((n_sel >= K) & (n_sel <= KPAD)
          & (kept_valid - extras >= MAX_OUT))

    def fast(_):
        fs, fi = lax.top_k(masked, MAX_OUT)
        fb = jnp.take(cols[:, :4], fi, axis=0)
        return fb, fs

    def slow(_):
        top_scores, order = lax.top_k(scores, K)
        top_boxes = jnp.take(boxes, order, axis=0)
        sp = jnp.concatenate(
            [top_scores, jnp.full((KPAD - K,), -1e9, jnp.float32)])
        bp = jnp.concatenate(
            [top_boxes, jnp.zeros((KPAD - K, 4), jnp.float32)])
        cols2 = jnp.concatenate([bp, sp[:, None]], axis=1)
        m2, _ = _nms_masked(cols2, cols2.T,
                            jnp.full((1, 1), K, jnp.int32))
        s2, i2 = lax.top_k(m2[0], MAX_OUT)
        b2 = jnp.take(bp, i2, axis=0)
        return b2, s2

    return lax.cond(ok, fast, slow, None)


# speedup vs baseline: 1.1666x; 1.1566x over previous
import jax
import jax.numpy as jnp
from jax import lax
from jax.experimental import pallas as pl
from jax.experimental.pallas import tpu as pltpu
from jax.experimental.pallas import tpu_sc as plsc

N = 20000
NPAD = 20480
NW = 16
CH = NPAD // NW
SLOT = 256
K = 2000
KPAD = 2560
T0 = 0.887
IOU_THRESH = 0.7
MAX_OUT = 300
MERGED = NW * SLOT


def _sc_select_body(s_hbm, x1_hbm, y1_hbm, x2_hbm, y2_hbm,
                    o_rows, o_cnt,
                    ch, co, vec16, sh, sh_cnt, st0, bufr, cl, sem):
    cid = lax.axis_index("c")
    wid = lax.axis_index("s")

    @pl.when(cid == 0)
    def _():
        base = wid * CH
        in_refs = (s_hbm, x1_hbm, y1_hbm, x2_hbm, y2_hbm)
        descs = [pltpu.async_copy(in_refs[p].at[pl.ds(base, CH)],
                                  ch[p], sem) for p in range(5)]
        for d in descs:
            d.wait()

        lane = lax.iota(jnp.int32, 16)

        def body(i, off):
            sv = ch[0][pl.ds(i * 16, 16)]
            m = sv > T0
            mi = m.astype(jnp.int32)
            rank = plsc.cumsum(mi) - mi
            idx = jnp.where(m, off + rank, CH + lane)
            plsc.store_scatter(co[0], [idx], sv)
            for p in range(1, 5):
                plsc.store_scatter(co[p], [idx], ch[p][pl.ds(i * 16, 16)])
            return off + jnp.max(plsc.all_reduce_population_count(m))

        cnt_w = lax.fori_loop(0, CH // 16, body, jnp.int32(0))

        for p in range(5):
            pltpu.sync_copy(co[p].at[pl.ds(0, SLOT)],
                            sh[p].at[pl.ds(wid * SLOT, SLOT)])
        vec16[...] = jnp.broadcast_to(cnt_w, (16,))
        pltpu.sync_copy(vec16, sh_cnt.at[pl.ds(wid * 16, 16)])
        plsc.subcore_barrier()

        for p in range(5):
            @pl.when(wid == p)
            def _merge(p=p):
                pltpu.sync_copy(sh_cnt, cl)
                pltpu.sync_copy(sh[p], st0)
                fill = -1e9 if p == 0 else 0.0

                def initr(g, _):
                    bufr[pl.ds(g * 16, 16)] = jnp.full((16,), fill,
                                                       jnp.float32)
                    return 0

                lax.fori_loop(0, KPAD // 16, initr, 0)

                def mbody(v, off):
                    cv = jnp.minimum(jnp.max(cl[pl.ds(v * 16, 16)]),
                                     jnp.int32(SLOT))

                    def inner(g, off2):
                        lane2 = lax.iota(jnp.int32, 16)
                        m2 = (lane2 + g * 16) < cv
                        mi2 = m2.astype(jnp.int32)
                        rank2 = plsc.cumsum(mi2) - mi2
                        pos = off2 + rank2
                        m3 = m2 & (pos < KPAD)
                        posr = jnp.where(m3, pos, MERGED + lane2)
                        x = st0[pl.ds(v * SLOT + g * 16, 16)]
                        plsc.store_scatter(bufr, [posr], x)
                        return off2 + jnp.max(
                            plsc.all_reduce_population_count(m2))

                    return lax.fori_loop(0, SLOT // 16, inner, off)

                lax.fori_loop(0, NW, mbody, jnp.int32(0))
                pltpu.sync_copy(bufr.at[pl.ds(0, KPAD)],
                                o_rows.at[pl.ds(((p + 4) % 5) * KPAD,
                                                KPAD)])

                if p == 0:
                    def cbody(v, c):
                        tot, ovf = c
                        cv = jnp.max(cl[pl.ds(v * 16, 16)])
                        return tot + cv, ovf | (cv > SLOT)

                    tot, ovf = lax.fori_loop(
                        0, NW, cbody, (jnp.int32(0), jnp.bool_(False)))
                    n_out = jnp.where(ovf, jnp.int32(99999), tot)
                    vec16[...] = jnp.broadcast_to(n_out, (16,))
                    pltpu.sync_copy(vec16, o_cnt)


def _sc_select(scores_p, x1, y1, x2, y2):
    mesh = plsc.VectorSubcoreMesh(core_axis_name="c", subcore_axis_name="s")
    f32 = jnp.float32
    kern = pl.kernel(
        lambda *refs: _sc_select_body(
            *refs[:7],
            ch=refs[7:12], co=refs[12:17], vec16=refs[17],
            sh=refs[18:23], sh_cnt=refs[23], st0=refs[24],
            bufr=refs[25], cl=refs[26], sem=refs[27]),
        out_type=[jax.ShapeDtypeStruct((5 * KPAD,), f32),
                  jax.ShapeDtypeStruct((16,), jnp.int32)],
        mesh=mesh,
        compiler_params=pltpu.CompilerParams(needs_layout_passes=False),
        scratch_types=(
            [pltpu.VMEM((CH,), f32)] * 5
            + [pltpu.VMEM((CH + 16,), f32)] * 5
            + [pltpu.VMEM((16,), jnp.int32)]
            + [pltpu.VMEM_SHARED((NW * SLOT,), f32)] * 5
            + [pltpu.VMEM_SHARED((NW * 16,), jnp.int32)]
            + [pltpu.VMEM((NW * SLOT,), f32)]
            + [pltpu.VMEM((MERGED + 16,), f32)]
            + [pltpu.VMEM((NW * 16,), jnp.int32)]
            + [pltpu.SemaphoreType.DMA]
        ),
    )
    return kern(scores_p, x1, y1, x2, y2)


RCH = 320


def _nms_kernel(cols_ref, rows_ref, nsel_ref, masked_ref, stats_ref, m_ref):
    x1r = rows_ref[0:1, :]
    y1r = rows_ref[1:2, :]
    x2r = rows_ref[2:3, :]
    y2r = rows_ref[3:4, :]
    sr = rows_ref[4:5, :]
    area_r = (x2r - x1r) * (y2r - y1r)

    for c in range(KPAD // RCH):
        tile = cols_ref[pl.ds(c * RCH, RCH), :]
        x1c = tile[:, 0:1]
        y1c = tile[:, 1:2]
        x2c = tile[:, 2:3]
        y2c = tile[:, 3:4]
        sc = tile[:, 4:5]
        area_c = (x2c - x1c) * (y2c - y1c)
        xx1 = jnp.maximum(x1c, x1r)
        yy1 = jnp.maximum(y1c, y1r)
        xx2 = jnp.minimum(x2c, x2r)
        yy2 = jnp.minimum(y2c, y2r)
        iw = jnp.clip(xx2 - xx1, 0.0)
        ih = jnp.clip(yy2 - yy1, 0.0)
        inter = iw * ih
        union = area_c + area_r - inter
        iou = inter / jnp.maximum(union, 1e-9)

        gi = lax.broadcasted_iota(jnp.int32, (RCH, KPAD), 0) + c * RCH
        gj = lax.broadcasted_iota(jnp.int32, (RCH, KPAD), 1)
        pri = (sc > sr) | ((sc == sr) & (gi < gj))
        m_ref[pl.ds(c * RCH, RCH), :] = (
            (iou > IOU_THRESH) & pri).astype(jnp.bfloat16)

    def cond(c):
        return c[1]

    def body(c):
        k, _ = c
        cnt = lax.dot_general(
            k.astype(jnp.bfloat16), m_ref[:, :],
            (((1,), (0,)), ((), ())),
            preferred_element_type=jnp.float32,
        )
        k_new = (cnt == 0.0).astype(jnp.float32)
        return k_new, jnp.any(k_new != k)

    k0 = jnp.ones((1, KPAD), jnp.float32)
    k_final, _ = lax.while_loop(cond, body, (k0, True))

    masked_ref[:, :] = jnp.where(k_final > 0.5, sr, -1e9)
    nsel = nsel_ref[0, 0]
    gcol = lax.broadcasted_iota(jnp.int32, (1, KPAD), 1)
    kept_valid = jnp.sum(jnp.where((k_final > 0.5) & (gcol < nsel),
                                   1.0, 0.0))
    stats_ref[:, :] = jnp.full((1, 128), kept_valid, jnp.float32)


def _nms_masked(cols, rows, nsel):
    return pl.pallas_call(
        _nms_kernel,
        out_shape=[jax.ShapeDtypeStruct((1, KPAD), jnp.float32),
                   jax.ShapeDtypeStruct((1, 128), jnp.float32)],
        scratch_shapes=[pltpu.VMEM((KPAD, KPAD), jnp.bfloat16)],
    )(cols, rows, nsel)


def kernel(boxes, scores):
    scores_p = jnp.concatenate(
        [scores, jnp.full((NPAD - N,), -1.0, jnp.float32)])
    bpad = jnp.concatenate([boxes, jnp.zeros((NPAD - N, 4), jnp.float32)])
    rows_f, n_sel_v = _sc_select(
        scores_p, bpad[:, 0], bpad[:, 1], bpad[:, 2], bpad[:, 3])
    n_sel = n_sel_v[0]
    rows = rows_f.reshape(5, KPAD)
    cols = rows.T

    masked_f, stats = _nms_masked(cols, rows,
                                  n_sel_v[:1].reshape(1, 1))
    masked = masked_f[0]
    kept_valid = stats[0, 0].astype(jnp.int32)

    extras = n_sel - K
    ok = ((n_sel >= K) & (n_sel <= KPAD)
          & (kept_valid - extras >= MAX_OUT))

    def fast(_):
        fs, fi = lax.top_k(masked, MAX_OUT)
        fb = jnp.take(cols[:, :4], fi, axis=0)
        return fb, fs

    def slow(_):
        top_scores, order = lax.top_k(scores, K)
        top_boxes = jnp.take(boxes, order, axis=0)
        sp = jnp.concatenate(
            [top_scores, jnp.full((KPAD - K,), -1e9, jnp.float32)])
        bp = jnp.concatenate(
            [top_boxes, jnp.zeros((KPAD - K, 4), jnp.float32)])
        cols2 = jnp.concatenate([bp, sp[:, None]], axis=1)
        m2, _ = _nms_masked(cols2, cols2.T,
                            jnp.full((1, 1), K, jnp.int32))
        s2, i2 = lax.top_k(m2[0], MAX_OUT)
        b2 = jnp.take(bp, i2, axis=0)
        return b2, s2

    return lax.cond(ok, fast, slow, None)


# speedup vs baseline: 1.3064x; 1.1198x over previous
import jax
import jax.numpy as jnp
from jax import lax
from jax.experimental import pallas as pl
from jax.experimental.pallas import tpu as pltpu
from jax.experimental.pallas import tpu_sc as plsc

N = 20000
NPAD = 20480
NW = 16
CH = NPAD // NW
SLOT = 256
K = 2000
KPAD = 2304
T0 = 0.8924
IOU_THRESH = 0.7
MAX_OUT = 300
MERGED = NW * SLOT


def _sc_select_body(s_hbm, x1_hbm, y1_hbm, x2_hbm, y2_hbm,
                    o_rows, o_cnt,
                    ch, co, vec16, sh, sh_cnt, st0, bufr, cl, sem):
    cid = lax.axis_index("c")
    wid = lax.axis_index("s")

    @pl.when(cid == 0)
    def _():
        base = wid * CH
        in_refs = (s_hbm, x1_hbm, y1_hbm, x2_hbm, y2_hbm)
        descs = [pltpu.async_copy(in_refs[p].at[pl.ds(base, CH)],
                                  ch[p], sem) for p in range(5)]
        for d in descs:
            d.wait()

        lane = lax.iota(jnp.int32, 16)

        def body(i, off):
            sv = ch[0][pl.ds(i * 16, 16)]
            m = sv > T0
            mi = m.astype(jnp.int32)
            rank = plsc.cumsum(mi) - mi
            idx = jnp.where(m, off + rank, CH + lane)
            plsc.store_scatter(co[0], [idx], sv)
            for p in range(1, 5):
                plsc.store_scatter(co[p], [idx], ch[p][pl.ds(i * 16, 16)])
            return off + jnp.max(plsc.all_reduce_population_count(m))

        cnt_w = lax.fori_loop(0, CH // 16, body, jnp.int32(0))

        for p in range(5):
            pltpu.sync_copy(co[p].at[pl.ds(0, SLOT)],
                            sh[p].at[pl.ds(wid * SLOT, SLOT)])
        vec16[...] = jnp.broadcast_to(cnt_w, (16,))
        pltpu.sync_copy(vec16, sh_cnt.at[pl.ds(wid * 16, 16)])
        plsc.subcore_barrier()

        for p in range(5):
            @pl.when(wid == p)
            def _merge(p=p):
                pltpu.sync_copy(sh_cnt, cl)
                pltpu.sync_copy(sh[p], st0)
                fill = -1e9 if p == 0 else 0.0

                def initr(g, _):
                    bufr[pl.ds(g * 16, 16)] = jnp.full((16,), fill,
                                                       jnp.float32)
                    return 0

                lax.fori_loop(0, KPAD // 16, initr, 0)

                def mbody(v, off):
                    cv = jnp.minimum(jnp.max(cl[pl.ds(v * 16, 16)]),
                                     jnp.int32(SLOT))

                    def inner(g, off2):
                        lane2 = lax.iota(jnp.int32, 16)
                        m2 = (lane2 + g * 16) < cv
                        mi2 = m2.astype(jnp.int32)
                        rank2 = plsc.cumsum(mi2) - mi2
                        pos = off2 + rank2
                        m3 = m2 & (pos < KPAD)
                        posr = jnp.where(m3, pos, MERGED + lane2)
                        x = st0[pl.ds(v * SLOT + g * 16, 16)]
                        plsc.store_scatter(bufr, [posr], x)
                        return off2 + jnp.max(
                            plsc.all_reduce_population_count(m2))

                    return lax.fori_loop(0, SLOT // 16, inner, off)

                lax.fori_loop(0, NW, mbody, jnp.int32(0))
                pltpu.sync_copy(bufr.at[pl.ds(0, KPAD)],
                                o_rows.at[pl.ds(((p + 4) % 5) * KPAD,
                                                KPAD)])

                if p == 0:
                    def cbody(v, c):
                        tot, ovf = c
                        cv = jnp.max(cl[pl.ds(v * 16, 16)])
                        return tot + cv, ovf | (cv > SLOT)

                    tot, ovf = lax.fori_loop(
                        0, NW, cbody, (jnp.int32(0), jnp.bool_(False)))
                    n_out = jnp.where(ovf, jnp.int32(99999), tot)
                    vec16[...] = jnp.broadcast_to(n_out, (16,))
                    pltpu.sync_copy(vec16, o_cnt)


def _sc_select(scores_p, x1, y1, x2, y2):
    mesh = plsc.VectorSubcoreMesh(core_axis_name="c", subcore_axis_name="s")
    f32 = jnp.float32
    kern = pl.kernel(
        lambda *refs: _sc_select_body(
            *refs[:7],
            ch=refs[7:12], co=refs[12:17], vec16=refs[17],
            sh=refs[18:23], sh_cnt=refs[23], st0=refs[24],
            bufr=refs[25], cl=refs[26], sem=refs[27]),
        out_type=[jax.ShapeDtypeStruct((5 * KPAD,), f32),
                  jax.ShapeDtypeStruct((16,), jnp.int32)],
        mesh=mesh,
        compiler_params=pltpu.CompilerParams(needs_layout_passes=False),
        scratch_types=(
            [pltpu.VMEM((CH,), f32)] * 5
            + [pltpu.VMEM((CH + 16,), f32)] * 5
            + [pltpu.VMEM((16,), jnp.int32)]
            + [pltpu.VMEM_SHARED((NW * SLOT,), f32)] * 5
            + [pltpu.VMEM_SHARED((NW * 16,), jnp.int32)]
            + [pltpu.VMEM((NW * SLOT,), f32)]
            + [pltpu.VMEM((MERGED + 16,), f32)]
            + [pltpu.VMEM((NW * 16,), jnp.int32)]
            + [pltpu.SemaphoreType.DMA]
        ),
    )
    return kern(scores_p, x1, y1, x2, y2)


RCH = 256


def _nms_kernel(cols_ref, rows_ref, nsel_ref, masked_ref, stats_ref, m_ref):
    x1r = rows_ref[0:1, :]
    y1r = rows_ref[1:2, :]
    x2r = rows_ref[2:3, :]
    y2r = rows_ref[3:4, :]
    sr = rows_ref[4:5, :]
    area_r = (x2r - x1r) * (y2r - y1r)

    for c in range(KPAD // RCH):
        tile = cols_ref[pl.ds(c * RCH, RCH), :]
        x1c = tile[:, 0:1]
        y1c = tile[:, 1:2]
        x2c = tile[:, 2:3]
        y2c = tile[:, 3:4]
        sc = tile[:, 4:5]
        area_c = (x2c - x1c) * (y2c - y1c)
        xx1 = jnp.maximum(x1c, x1r)
        yy1 = jnp.maximum(y1c, y1r)
        xx2 = jnp.minimum(x2c, x2r)
        yy2 = jnp.minimum(y2c, y2r)
        iw = jnp.clip(xx2 - xx1, 0.0)
        ih = jnp.clip(yy2 - yy1, 0.0)
        inter = iw * ih
        union = area_c + area_r - inter
        iou = inter / jnp.maximum(union, 1e-9)

        gi = lax.broadcasted_iota(jnp.int32, (RCH, KPAD), 0) + c * RCH
        gj = lax.broadcasted_iota(jnp.int32, (RCH, KPAD), 1)
        pri = (sc > sr) | ((sc == sr) & (gi < gj))
        m_ref[pl.ds(c * RCH, RCH), :] = (
            (iou > IOU_THRESH) & pri).astype(jnp.bfloat16)

    def cond(c):
        return c[1]

    def body(c):
        k, _ = c
        cnt = lax.dot_general(
            k.astype(jnp.bfloat16), m_ref[:, :],
            (((1,), (0,)), ((), ())),
            preferred_element_type=jnp.float32,
        )
        k_new = (cnt == 0.0).astype(jnp.float32)
        return k_new, jnp.any(k_new != k)

    k0 = jnp.ones((1, KPAD), jnp.float32)
    k_final, _ = lax.while_loop(cond, body, (k0, True))

    masked_ref[:, :] = jnp.where(k_final > 0.5, sr, -1e9)
    nsel = nsel_ref[0, 0]
    gcol = lax.broadcasted_iota(jnp.int32, (1, KPAD), 1)
    kept_valid = jnp.sum(jnp.where((k_final > 0.5) & (gcol < nsel),
                                   1.0, 0.0))
    stats_ref[:, :] = jnp.full((1, 128), kept_valid, jnp.float32)


def _nms_masked(cols, rows, nsel):
    return pl.pallas_call(
        _nms_kernel,
        out_shape=[jax.ShapeDtypeStruct((1, KPAD), jnp.float32),
                   jax.ShapeDtypeStruct((1, 128), jnp.float32)],
        scratch_shapes=[pltpu.VMEM((KPAD, KPAD), jnp.bfloat16)],
    )(cols, rows, nsel)


def kernel(boxes, scores):
    scores_p = jnp.concatenate(
        [scores, jnp.full((NPAD - N,), -1.0, jnp.float32)])
    bpad = jnp.concatenate([boxes, jnp.zeros((NPAD - N, 4), jnp.float32)])
    rows_f, n_sel_v = _sc_select(
        scores_p, bpad[:, 0], bpad[:, 1], bpad[:, 2], bpad[:, 3])
    n_sel = n_sel_v[0]
    rows = rows_f.reshape(5, KPAD)
    cols = rows.T

    masked_f, stats = _nms_masked(cols, rows,
                                  n_sel_v[:1].reshape(1, 1))
    masked = masked_f[0]
    kept_valid = stats[0, 0].astype(jnp.int32)

    extras = n_sel - K
    ok = ((n_sel >= K) & (n_sel <= KPAD)
          & (kept_valid - extras >= MAX_OUT))

    def fast(_):
        fs, fi = lax.top_k(masked, MAX_OUT)
        fb = jnp.take(cols[:, :4], fi, axis=0)
        return fb, fs

    def slow(_):
        top_scores, order = lax.top_k(scores, K)
        top_boxes = jnp.take(boxes, order, axis=0)
        sp = jnp.concatenate(
            [top_scores, jnp.full((KPAD - K,), -1e9, jnp.float32)])
        bp = jnp.concatenate(
            [top_boxes, jnp.zeros((KPAD - K, 4), jnp.float32)])
        cols2 = jnp.concatenate([bp, sp[:, None]], axis=1)
        m2, _ = _nms_masked(cols2, cols2.T,
                            jnp.full((1, 1), K, jnp.int32))
        s2, i2 = lax.top_k(m2[0], MAX_OUT)
        b2 = jnp.take(bp, i2, axis=0)
        return b2, s2

    return lax.cond(ok, fast, slow, None)


# speedup vs baseline: 1.3080x; 1.0012x over previous
import jax
import jax.numpy as jnp
from jax import lax
from jax.experimental import pallas as pl
from jax.experimental.pallas import tpu as pltpu
from jax.experimental.pallas import tpu_sc as plsc

N = 20000
NPAD = 20480
NW = 16
CH = NPAD // NW
SLOT = 256
K = 2000
KPAD = 2304
T0 = 0.8924
IOU_THRESH = 0.7
MAX_OUT = 300
MERGED = NW * SLOT


def _sc_select_body(s_hbm, x1_hbm, y1_hbm, x2_hbm, y2_hbm,
                    o_rows, o_cnt,
                    ch, co, vec16, sh, sh_cnt, st0, bufr, cl, sem):
    cid = lax.axis_index("c")
    wid = lax.axis_index("s")

    @pl.when(cid == 0)
    def _():
        base = wid * CH
        in_refs = (s_hbm, x1_hbm, y1_hbm, x2_hbm, y2_hbm)
        descs = [pltpu.async_copy(in_refs[p].at[pl.ds(base, CH)],
                                  ch[p], sem) for p in range(5)]
        for d in descs:
            d.wait()

        lane = lax.iota(jnp.int32, 16)

        def body(i, off):
            sv = ch[0][pl.ds(i * 16, 16)]
            m = sv > T0
            mi = m.astype(jnp.int32)
            rank = plsc.cumsum(mi) - mi
            idx = jnp.where(m, off + rank, CH + lane)
            plsc.store_scatter(co[0], [idx], sv)
            for p in range(1, 5):
                plsc.store_scatter(co[p], [idx], ch[p][pl.ds(i * 16, 16)])
            return off + jnp.max(plsc.all_reduce_population_count(m))

        cnt_w = lax.fori_loop(0, CH // 16, body, jnp.int32(0))

        for p in range(5):
            pltpu.sync_copy(co[p].at[pl.ds(0, SLOT)],
                            sh[p].at[pl.ds(wid * SLOT, SLOT)])
        vec16[...] = jnp.broadcast_to(cnt_w, (16,))
        pltpu.sync_copy(vec16, sh_cnt.at[pl.ds(wid * 16, 16)])
        plsc.subcore_barrier()

        for p in range(5):
            @pl.when(wid == p)
            def _merge(p=p):
                pltpu.sync_copy(sh_cnt, cl)
                pltpu.sync_copy(sh[p], st0)
                fill = -1e9 if p == 0 else 0.0

                def initr(g, _):
                    bufr[pl.ds(g * 16, 16)] = jnp.full((16,), fill,
                                                       jnp.float32)
                    return 0

                lax.fori_loop(0, KPAD // 16, initr, 0)

                def mbody(v, off):
                    cv = jnp.minimum(jnp.max(cl[pl.ds(v * 16, 16)]),
                                     jnp.int32(SLOT))

                    def inner(g, off2):
                        lane2 = lax.iota(jnp.int32, 16)
                        m2 = (lane2 + g * 16) < cv
                        mi2 = m2.astype(jnp.int32)
                        rank2 = plsc.cumsum(mi2) - mi2
                        pos = off2 + rank2
                        m3 = m2 & (pos < KPAD)
                        posr = jnp.where(m3, pos, MERGED + lane2)
                        x = st0[pl.ds(v * SLOT + g * 16, 16)]
                        plsc.store_scatter(bufr, [posr], x)
                        return off2 + jnp.max(
                            plsc.all_reduce_population_count(m2))

                    return lax.fori_loop(0, SLOT // 16, inner, off)

                lax.fori_loop(0, NW, mbody, jnp.int32(0))
                pltpu.sync_copy(bufr.at[pl.ds(0, KPAD)],
                                o_rows.at[pl.ds(((p + 4) % 5) * KPAD,
                                                KPAD)])

                if p == 0:
                    def cbody(v, c):
                        tot, ovf = c
                        cv = jnp.max(cl[pl.ds(v * 16, 16)])
                        return tot + cv, ovf | (cv > SLOT)

                    tot, ovf = lax.fori_loop(
                        0, NW, cbody, (jnp.int32(0), jnp.bool_(False)))
                    n_out = jnp.where(ovf, jnp.int32(99999), tot)
                    vec16[...] = jnp.broadcast_to(n_out, (16,))
                    pltpu.sync_copy(vec16, o_cnt)


def _sc_select(scores_p, x1, y1, x2, y2):
    mesh = plsc.VectorSubcoreMesh(core_axis_name="c", subcore_axis_name="s")
    f32 = jnp.float32
    kern = pl.kernel(
        lambda *refs: _sc_select_body(
            *refs[:7],
            ch=refs[7:12], co=refs[12:17], vec16=refs[17],
            sh=refs[18:23], sh_cnt=refs[23], st0=refs[24],
            bufr=refs[25], cl=refs[26], sem=refs[27]),
        out_type=[jax.ShapeDtypeStruct((5 * KPAD,), f32),
                  jax.ShapeDtypeStruct((16,), jnp.int32)],
        mesh=mesh,
        compiler_params=pltpu.CompilerParams(needs_layout_passes=False),
        scratch_types=(
            [pltpu.VMEM((CH,), f32)] * 5
            + [pltpu.VMEM((CH + 16,), f32)] * 5
            + [pltpu.VMEM((16,), jnp.int32)]
            + [pltpu.VMEM_SHARED((NW * SLOT,), f32)] * 5
            + [pltpu.VMEM_SHARED((NW * 16,), jnp.int32)]
            + [pltpu.VMEM((NW * SLOT,), f32)]
            + [pltpu.VMEM((MERGED + 16,), f32)]
            + [pltpu.VMEM((NW * 16,), jnp.int32)]
            + [pltpu.SemaphoreType.DMA]
        ),
    )
    return kern(scores_p, x1, y1, x2, y2)


RCH = 256


def _nms_kernel(cols_ref, rows_ref, nsel_ref, masked_ref, stats_ref, m_ref):
    x1r = rows_ref[0:1, :]
    y1r = rows_ref[1:2, :]
    x2r = rows_ref[2:3, :]
    y2r = rows_ref[3:4, :]
    sr = rows_ref[4:5, :]
    area_r = (x2r - x1r) * (y2r - y1r)

    for c in range(KPAD // RCH):
        tile = cols_ref[pl.ds(c * RCH, RCH), :]
        x1c = tile[:, 0:1]
        y1c = tile[:, 1:2]
        x2c = tile[:, 2:3]
        y2c = tile[:, 3:4]
        sc = tile[:, 4:5]
        area_c = (x2c - x1c) * (y2c - y1c)
        xx1 = jnp.maximum(x1c, x1r)
        yy1 = jnp.maximum(y1c, y1r)
        xx2 = jnp.minimum(x2c, x2r)
        yy2 = jnp.minimum(y2c, y2r)
        iw = jnp.clip(xx2 - xx1, 0.0)
        ih = jnp.clip(yy2 - yy1, 0.0)
        inter = iw * ih
        union = area_c + area_r - inter
        iou = inter / jnp.maximum(union, 1e-9)

        gi = lax.broadcasted_iota(jnp.int32, (RCH, KPAD), 0) + c * RCH
        gj = lax.broadcasted_iota(jnp.int32, (RCH, KPAD), 1)
        pri = (sc > sr) | ((sc == sr) & (gi < gj))
        m_ref[pl.ds(c * RCH, RCH), :] = (
            (iou > IOU_THRESH) & pri).astype(jnp.bfloat16)

    def cond(c):
        return c[1]

    def step(k):
        cnt = lax.dot_general(
            k.astype(jnp.bfloat16), m_ref[:, :],
            (((1,), (0,)), ((), ())),
            preferred_element_type=jnp.float32,
        )
        return (cnt == 0.0).astype(jnp.float32)

    def body(c):
        k, _ = c
        k1 = step(k)
        k2 = step(k1)
        return k2, jnp.any(k2 != k1)

    k0 = jnp.ones((1, KPAD), jnp.float32)
    k_final, _ = lax.while_loop(cond, body, (k0, True))

    masked_ref[:, :] = jnp.where(k_final > 0.5, sr, -1e9)
    nsel = nsel_ref[0, 0]
    gcol = lax.broadcasted_iota(jnp.int32, (1, KPAD), 1)
    kept_valid = jnp.sum(jnp.where((k_final > 0.5) & (gcol < nsel),
                                   1.0, 0.0))
    stats_ref[:, :] = jnp.full((1, 128), kept_valid, jnp.float32)


def _nms_masked(cols, rows, nsel):
    return pl.pallas_call(
        _nms_kernel,
        out_shape=[jax.ShapeDtypeStruct((1, KPAD), jnp.float32),
                   jax.ShapeDtypeStruct((1, 128), jnp.float32)],
        scratch_shapes=[pltpu.VMEM((KPAD, KPAD), jnp.bfloat16)],
    )(cols, rows, nsel)


def kernel(boxes, scores):
    scores_p = jnp.concatenate(
        [scores, jnp.full((NPAD - N,), -1.0, jnp.float32)])
    bpad = jnp.concatenate([boxes, jnp.zeros((NPAD - N, 4), jnp.float32)])
    rows_f, n_sel_v = _sc_select(
        scores_p, bpad[:, 0], bpad[:, 1], bpad[:, 2], bpad[:, 3])
    n_sel = n_sel_v[0]
    rows = rows_f.reshape(5, KPAD)
    cols = rows.T

    masked_f, stats = _nms_masked(cols, rows,
                                  n_sel_v[:1].reshape(1, 1))
    masked = masked_f[0]
    kept_valid = stats[0, 0].astype(jnp.int32)

    extras = n_sel - K
    ok = ((n_sel >= K) & (n_sel <= KPAD)
          & (kept_valid - extras >= MAX_OUT))

    def fast(_):
        fs, fi = lax.top_k(masked, MAX_OUT)
        fb = jnp.take(cols[:, :4], fi, axis=0)
        return fb, fs

    def slow(_):
        top_scores, order = lax.top_k(scores, K)
        top_boxes = jnp.take(boxes, order, axis=0)
        sp = jnp.concatenate(
            [top_scores, jnp.full((KPAD - K,), -1e9, jnp.float32)])
        bp = jnp.concatenate(
            [top_boxes, jnp.zeros((KPAD - K, 4), jnp.float32)])
        cols2 = jnp.concatenate([bp, sp[:, None]], axis=1)
        m2, _ = _nms_masked(cols2, cols2.T,
                            jnp.full((1, 1), K, jnp.int32))
        s2, i2 = lax.top_k(m2[0], MAX_OUT)
        b2 = jnp.take(bp, i2, axis=0)
        return b2, s2

    return lax.cond(ok, fast, slow, None)
